# baseline (speedup 1.0000x reference)
import jax
import jax.numpy as jnp
from jax.experimental import pallas as pl
from jax.experimental.pallas import tpu as pltpu

N_ROWS = 100000
N_COLS = 128

B_TC = 5000
NSTREAM = 10
G_TC = 2
assert NSTREAM * B_TC * G_TC == N_ROWS


def _tc_body(*refs):
    x_refs = refs[:NSTREAM]
    o_ref = refs[NSTREAM]
    acc_ref = refs[NSTREAM + 1]
    i = pl.program_id(0)

    @pl.when(i == 0)
    def _():
        acc_ref[...] = jnp.zeros_like(acc_ref)

    part = acc_ref[...]
    for x_ref in x_refs:
        part += jnp.sum(x_ref[...].reshape(B_TC // 8, 8, N_COLS), axis=0)
    acc_ref[...] = part

    @pl.when(i == G_TC - 1)
    def _():
        o_ref[...] = jnp.sum(acc_ref[...], axis=0, keepdims=True)


_tc_call = pl.pallas_call(
    _tc_body,
    grid=(G_TC,),
    in_specs=[
        pl.BlockSpec((B_TC, N_COLS), lambda i, _k=k: (i * NSTREAM + _k, 0))
        for k in range(NSTREAM)
    ],
    out_specs=pl.BlockSpec((1, N_COLS), lambda i: (0, 0)),
    out_shape=jax.ShapeDtypeStruct((1, N_COLS), jnp.float32),
    scratch_shapes=[pltpu.VMEM((8, N_COLS), jnp.float32)],
)


def kernel(x):
    return _tc_call(*([x] * NSTREAM))


# speedup vs baseline: 1.0181x; 1.0181x over previous
import jax
import jax.numpy as jnp
from jax.experimental import pallas as pl
from jax.experimental.pallas import tpu as pltpu

N_ROWS = 100000
N_COLS = 128

B_TC = 5000
NSTREAM = 5
G_TC = 4
assert NSTREAM * B_TC * G_TC == N_ROWS


def _tc_body(*refs):
    x_refs = refs[:NSTREAM]
    o_ref = refs[NSTREAM]
    acc_ref = refs[NSTREAM + 1]
    i = pl.program_id(0)

    @pl.when(i == 0)
    def _():
        acc_ref[...] = jnp.zeros_like(acc_ref)

    part = acc_ref[...]
    for x_ref in x_refs:
        part += jnp.sum(x_ref[...].reshape(B_TC // 8, 8, N_COLS), axis=0)
    acc_ref[...] = part

    @pl.when(i == G_TC - 1)
    def _():
        o_ref[...] = jnp.sum(acc_ref[...], axis=0, keepdims=True)


_tc_call = pl.pallas_call(
    _tc_body,
    grid=(G_TC,),
    in_specs=[
        pl.BlockSpec((B_TC, N_COLS), lambda i, _k=k: (i * NSTREAM + _k, 0))
        for k in range(NSTREAM)
    ],
    out_specs=pl.BlockSpec((1, N_COLS), lambda i: (0, 0)),
    out_shape=jax.ShapeDtypeStruct((1, N_COLS), jnp.float32),
    scratch_shapes=[pltpu.VMEM((8, N_COLS), jnp.float32)],
)


def kernel(x):
    return _tc_call(*([x] * NSTREAM))
